# Initial kernel scaffold; baseline (speedup 1.0000x reference)
import jax
import jax.numpy as jnp
from jax.experimental import pallas as pl


def kernel(z, embedding_weight):
    raise NotImplementedError("write your pallas kernel here")


# speedup vs baseline: 6.0180x; 6.0180x over previous
import functools

import jax
import jax.numpy as jnp
from jax.experimental import pallas as pl
from jax.experimental.pallas import tpu as pltpu

VOCAB = 8192
DIM = 32
BETA = 0.25
N = 8192
BLK = 128
NB = N // BLK
HALF = VOCAB // 2


def _vq_body(z_ref, sz_ref, w1_ref, w2_ref, sw_ref,
             enc_ref, zq_ref, idx_ref, loss_ref, perp_ref,
             hist_ref, lacc_ref):
    i = pl.program_id(0)

    @pl.when(i == 0)
    def _init():
        hist_ref[...] = jnp.zeros_like(hist_ref)
        lacc_ref[...] = jnp.zeros_like(lacc_ref)

    zb = z_ref[...]
    zh = zb.astype(jnp.bfloat16)
    mm = lambda a, b: jax.lax.dot_general(
        a, b, (((1,), (1,)), ((), ())), preferred_element_type=jnp.float32)
    m = mm(zh, w1_ref[...]) + mm(zh, w2_ref[...])
    d = (sz_ref[...] + sw_ref[...]) - 2.0 * m
    col = jax.lax.broadcasted_iota(jnp.int32, d.shape, 1)

    d1, d2 = d[:, :HALF], d[:, HALF:]
    c1, c2 = col[:, :HALF], col[:, HALF:]
    v1 = jnp.min(d1, axis=1)
    i1 = jnp.min(jnp.where(d1 == v1[:, None], c1, VOCAB), axis=1)
    v1b = v1.astype(jnp.bfloat16).astype(jnp.float32)
    v2 = jnp.min(d2, axis=1)
    i2 = jnp.min(jnp.where(d2 == v2[:, None], c2, VOCAB), axis=1)
    idx = jnp.where(v1b <= v2, i1, i2)

    hot = col == idx[:, None]
    onehot = hot.astype(jnp.float32)
    enc_ref[...] = onehot
    ob = hot.astype(jnp.bfloat16)
    gm = lambda a, b: jax.lax.dot_general(
        a, b, (((1,), (0,)), ((), ())), preferred_element_type=jnp.float32)
    zq = gm(ob, w1_ref[...]) + gm(ob, w2_ref[...])
    zq_ref[...] = zb + (zq - zb)
    idx_ref[...] = idx.reshape(1, 1, BLK)
    hist_ref[...] += jnp.sum(onehot, axis=0, keepdims=True)
    lacc_ref[...] += jnp.sum((zq - zb) ** 2).reshape(1, 1)

    @pl.when(i == NB - 1)
    def _fin():
        e = hist_ref[...] / float(N)
        perp_ref[...] = jnp.exp(-jnp.sum(e * jnp.log(e + 1e-10))).reshape(1, 1)
        mse = lacc_ref[0, 0] / float(N * DIM)
        loss_ref[...] = (mse + BETA * mse).reshape(1, 1)


@jax.jit
def kernel(z, embedding_weight):
    W = embedding_weight
    zp = jnp.transpose(z, (0, 2, 1))
    z_flat = zp.reshape(-1, W.shape[1])
    s_z = jnp.sum(z_flat ** 2, axis=1, keepdims=True)
    s_w = jnp.sum(W ** 2, axis=1).reshape(1, VOCAB)
    w1 = W.astype(jnp.bfloat16)
    w2 = (W - w1.astype(jnp.float32)).astype(jnp.bfloat16)
    z_flat, s_z, w1, w2, s_w = jax.lax.optimization_barrier(
        (z_flat, s_z, w1, w2, s_w))

    enc, zq_st, idx3, loss, perp = pl.pallas_call(
        _vq_body,
        grid=(NB,),
        in_specs=[
            pl.BlockSpec((BLK, DIM), lambda i: (i, 0)),
            pl.BlockSpec((BLK, 1), lambda i: (i, 0)),
            pl.BlockSpec((VOCAB, DIM), lambda i: (0, 0)),
            pl.BlockSpec((VOCAB, DIM), lambda i: (0, 0)),
            pl.BlockSpec((1, VOCAB), lambda i: (0, 0)),
        ],
        out_specs=[
            pl.BlockSpec((BLK, VOCAB), lambda i: (i, 0)),
            pl.BlockSpec((BLK, DIM), lambda i: (i, 0)),
            pl.BlockSpec((1, 1, BLK), lambda i: (i, 0, 0)),
            pl.BlockSpec((1, 1), lambda i: (0, 0)),
            pl.BlockSpec((1, 1), lambda i: (0, 0)),
        ],
        out_shape=[
            jax.ShapeDtypeStruct((N, VOCAB), jnp.float32),
            jax.ShapeDtypeStruct((N, DIM), jnp.float32),
            jax.ShapeDtypeStruct((NB, 1, BLK), jnp.int32),
            jax.ShapeDtypeStruct((1, 1), jnp.float32),
            jax.ShapeDtypeStruct((1, 1), jnp.float32),
        ],
        scratch_shapes=[
            pltpu.VMEM((1, VOCAB), jnp.float32),
            pltpu.VMEM((1, 1), jnp.float32),
        ],
    )(z_flat, s_z, w1, w2, s_w)

    z_q_st = zq_st.reshape(zp.shape)
    min_idx = idx3.reshape(zp.shape[0], -1)
    return (loss[0, 0], z_q_st, perp[0, 0], enc, min_idx)


# speedup vs baseline: 6.2587x; 1.0400x over previous
import functools

import jax
import jax.numpy as jnp
from jax.experimental import pallas as pl
from jax.experimental.pallas import tpu as pltpu

VOCAB = 8192
DIM = 32
BETA = 0.25
N = 8192
BLK = 256
NB = N // BLK
HALF = VOCAB // 2


def _vq_body(z_ref, sz_ref, w1_ref, w2_ref, sw_ref,
             enc_ref, zq_ref, idx_ref, loss_ref, perp_ref,
             hist_ref, lacc_ref):
    i = pl.program_id(0)

    @pl.when(i == 0)
    def _init():
        hist_ref[...] = jnp.zeros_like(hist_ref)
        lacc_ref[...] = jnp.zeros_like(lacc_ref)

    zb = z_ref[...]
    zh = zb.astype(jnp.bfloat16)
    mm = lambda a, b: jax.lax.dot_general(
        a, b, (((1,), (1,)), ((), ())), preferred_element_type=jnp.float32)
    m = mm(zh, w1_ref[...]) + mm(zh, w2_ref[...])
    d = (sz_ref[...] + sw_ref[...]) - 2.0 * m
    col = jax.lax.broadcasted_iota(jnp.int32, d.shape, 1)

    d1, d2 = d[:, :HALF], d[:, HALF:]
    c1, c2 = col[:, :HALF], col[:, HALF:]
    v1 = jnp.min(d1, axis=1)
    i1 = jnp.min(jnp.where(d1 == v1[:, None], c1, VOCAB), axis=1)
    v1b = v1.astype(jnp.bfloat16).astype(jnp.float32)
    v2 = jnp.min(d2, axis=1)
    i2 = jnp.min(jnp.where(d2 == v2[:, None], c2, VOCAB), axis=1)
    idx = jnp.where(v1b <= v2, i1, i2)

    hot = col == idx[:, None]
    onehot = hot.astype(jnp.float32)
    enc_ref[...] = onehot
    ob = hot.astype(jnp.bfloat16)
    gm = lambda a, b: jax.lax.dot_general(
        a, b, (((1,), (0,)), ((), ())), preferred_element_type=jnp.float32)
    zq = gm(ob, w1_ref[...]) + gm(ob, w2_ref[...])
    zq_ref[...] = zb + (zq - zb)
    idx_ref[...] = idx.reshape(1, 1, BLK)
    hist_ref[...] += jnp.sum(onehot, axis=0, keepdims=True)
    lacc_ref[...] += jnp.sum((zq - zb) ** 2).reshape(1, 1)

    @pl.when(i == NB - 1)
    def _fin():
        e = hist_ref[...] / float(N)
        perp_ref[...] = jnp.exp(-jnp.sum(e * jnp.log(e + 1e-10))).reshape(1, 1)
        mse = lacc_ref[0, 0] / float(N * DIM)
        loss_ref[...] = (mse + BETA * mse).reshape(1, 1)


@jax.jit
def kernel(z, embedding_weight):
    W = embedding_weight
    zp = jnp.transpose(z, (0, 2, 1))
    z_flat = zp.reshape(-1, W.shape[1])
    s_z = jnp.sum(z_flat ** 2, axis=1, keepdims=True)
    s_w = jnp.sum(W ** 2, axis=1).reshape(1, VOCAB)
    w1 = W.astype(jnp.bfloat16)
    w2 = (W - w1.astype(jnp.float32)).astype(jnp.bfloat16)
    z_flat, s_z, w1, w2, s_w = jax.lax.optimization_barrier(
        (z_flat, s_z, w1, w2, s_w))

    enc, zq_st, idx3, loss, perp = pl.pallas_call(
        _vq_body,
        grid=(NB,),
        in_specs=[
            pl.BlockSpec((BLK, DIM), lambda i: (i, 0)),
            pl.BlockSpec((BLK, 1), lambda i: (i, 0)),
            pl.BlockSpec((VOCAB, DIM), lambda i: (0, 0)),
            pl.BlockSpec((VOCAB, DIM), lambda i: (0, 0)),
            pl.BlockSpec((1, VOCAB), lambda i: (0, 0)),
        ],
        out_specs=[
            pl.BlockSpec((BLK, VOCAB), lambda i: (i, 0)),
            pl.BlockSpec((BLK, DIM), lambda i: (i, 0)),
            pl.BlockSpec((1, 1, BLK), lambda i: (i, 0, 0)),
            pl.BlockSpec((1, 1), lambda i: (0, 0)),
            pl.BlockSpec((1, 1), lambda i: (0, 0)),
        ],
        out_shape=[
            jax.ShapeDtypeStruct((N, VOCAB), jnp.float32),
            jax.ShapeDtypeStruct((N, DIM), jnp.float32),
            jax.ShapeDtypeStruct((NB, 1, BLK), jnp.int32),
            jax.ShapeDtypeStruct((1, 1), jnp.float32),
            jax.ShapeDtypeStruct((1, 1), jnp.float32),
        ],
        scratch_shapes=[
            pltpu.VMEM((1, VOCAB), jnp.float32),
            pltpu.VMEM((1, 1), jnp.float32),
        ],
    )(z_flat, s_z, w1, w2, s_w)

    z_q_st = zq_st.reshape(zp.shape)
    min_idx = idx3.reshape(zp.shape[0], -1)
    return (loss[0, 0], z_q_st, perp[0, 0], enc, min_idx)


# speedup vs baseline: 6.4436x; 1.0295x over previous
import jax
import jax.numpy as jnp
from jax.experimental import pallas as pl
from jax.experimental.pallas import tpu as pltpu

VOCAB = 8192
DIM = 32
BETA = 0.25
B = 8
SEQ = 1024
N = B * SEQ
BLK = 256
NSC = SEQ // BLK
HALF = VOCAB // 2


def _vq_body(z_ref, w1_ref, w2_ref, sw_ref,
             enc_ref, zq_ref, idx_ref, loss_ref, perp_ref,
             hist_ref, lacc_ref):
    b = pl.program_id(0)
    s = pl.program_id(1)

    @pl.when((b == 0) & (s == 0))
    def _init():
        hist_ref[...] = jnp.zeros_like(hist_ref)
        lacc_ref[...] = jnp.zeros_like(lacc_ref)

    zb = z_ref[...].reshape(DIM, BLK).T
    sz = jnp.sum(zb * zb, axis=1, keepdims=True)
    zh = zb.astype(jnp.bfloat16)
    mm = lambda a, b_: jax.lax.dot_general(
        a, b_, (((1,), (1,)), ((), ())), preferred_element_type=jnp.float32)
    m = mm(zh, w1_ref[...]) + mm(zh, w2_ref[...])
    d = (sz + sw_ref[...]) - 2.0 * m
    col = jax.lax.broadcasted_iota(jnp.int32, d.shape, 1)

    d1, d2 = d[:, :HALF], d[:, HALF:]
    c1, c2 = col[:, :HALF], col[:, HALF:]
    v1 = jnp.min(d1, axis=1)
    i1 = jnp.min(jnp.where(d1 == v1[:, None], c1, VOCAB), axis=1)
    v1b = v1.astype(jnp.bfloat16).astype(jnp.float32)
    v2 = jnp.min(d2, axis=1)
    i2 = jnp.min(jnp.where(d2 == v2[:, None], c2, VOCAB), axis=1)
    idx = jnp.where(v1b <= v2, i1, i2)

    hot = col == idx[:, None]
    onehot = hot.astype(jnp.float32)
    enc_ref[...] = onehot
    ob = hot.astype(jnp.bfloat16)
    gm = lambda a, b_: jax.lax.dot_general(
        a, b_, (((1,), (0,)), ((), ())), preferred_element_type=jnp.float32)
    zq = gm(ob, w1_ref[...]) + gm(ob, w2_ref[...])
    zq_ref[...] = zb + (zq - zb)
    idx_ref[...] = idx.reshape(1, 1, BLK)
    hist_ref[...] += jnp.sum(onehot, axis=0, keepdims=True)
    lacc_ref[...] += jnp.sum((zq - zb) ** 2).reshape(1, 1)

    @pl.when((b == B - 1) & (s == NSC - 1))
    def _fin():
        e = hist_ref[...] / float(N)
        perp_ref[...] = jnp.exp(-jnp.sum(e * jnp.log(e + 1e-10))).reshape(1, 1)
        mse = lacc_ref[0, 0] / float(N * DIM)
        loss_ref[...] = (mse + BETA * mse).reshape(1, 1)


@jax.jit
def kernel(z, embedding_weight):
    W = embedding_weight
    s_w = jnp.sum(W ** 2, axis=1).reshape(1, VOCAB)
    w1 = W.astype(jnp.bfloat16)
    w2 = (W - w1.astype(jnp.float32)).astype(jnp.bfloat16)
    w1, w2, s_w = jax.lax.optimization_barrier((w1, w2, s_w))

    enc, zq_st, idx3, loss, perp = pl.pallas_call(
        _vq_body,
        grid=(B, NSC),
        in_specs=[
            pl.BlockSpec((1, DIM, BLK), lambda b, s: (b, 0, s)),
            pl.BlockSpec((VOCAB, DIM), lambda b, s: (0, 0)),
            pl.BlockSpec((VOCAB, DIM), lambda b, s: (0, 0)),
            pl.BlockSpec((1, VOCAB), lambda b, s: (0, 0)),
        ],
        out_specs=[
            pl.BlockSpec((BLK, VOCAB), lambda b, s: (b * NSC + s, 0)),
            pl.BlockSpec((BLK, DIM), lambda b, s: (b * NSC + s, 0)),
            pl.BlockSpec((1, 1, BLK), lambda b, s: (b * NSC + s, 0, 0)),
            pl.BlockSpec((1, 1), lambda b, s: (0, 0)),
            pl.BlockSpec((1, 1), lambda b, s: (0, 0)),
        ],
        out_shape=[
            jax.ShapeDtypeStruct((N, VOCAB), jnp.float32),
            jax.ShapeDtypeStruct((N, DIM), jnp.float32),
            jax.ShapeDtypeStruct((N // BLK, 1, BLK), jnp.int32),
            jax.ShapeDtypeStruct((1, 1), jnp.float32),
            jax.ShapeDtypeStruct((1, 1), jnp.float32),
        ],
        scratch_shapes=[
            pltpu.VMEM((1, VOCAB), jnp.float32),
            pltpu.VMEM((1, 1), jnp.float32),
        ],
    )(z, w1, w2, s_w)

    z_q_st = zq_st.reshape(B, SEQ, DIM)
    min_idx = idx3.reshape(B, SEQ)
    return (loss[0, 0], z_q_st, perp[0, 0], enc, min_idx)
